# baseline (speedup 1.0000x reference)
import jax
import jax.numpy as jnp
from jax import lax
from jax.experimental import pallas as pl
from jax.experimental.pallas import tpu as pltpu
from jax.experimental.pallas import tpu_sc as plsc

NUM_BUCKETS = 32
MAX_DISTANCE = 128
HEADS = 16
S_I = 2048
S_J = 2048

BI = 8
TLANE = 4224
NBUF = 8
_LOG_DENOM = 2.0794415416798357

_SC_LANES = 16
_SC_CORES = 2
_HALF = TLANE // _SC_CORES
_CHUNKS = _HALF // _SC_LANES


def _bucketize_body(delta_ref, out_ref):
    k = lax.broadcasted_iota(jnp.int32, (1, TLANE), 1)
    rel = k - S_I + delta_ref[0]
    n = jnp.maximum(-rel, 0)
    max_exact = NUM_BUCKETS // 2
    is_small = n < max_exact
    safe_n = jnp.maximum(n, 1)
    val_if_large = max_exact + (
        jnp.log(safe_n.astype(jnp.float32) / max_exact)
        / _LOG_DENOM
        * (NUM_BUCKETS - max_exact)
    ).astype(jnp.int32)
    val_if_large = jnp.minimum(val_if_large, NUM_BUCKETS - 1)
    bucket = jnp.where(is_small, n, val_if_large)
    out_ref[:, :] = jnp.broadcast_to(bucket, (8, TLANE))


def _sc_gather_body(bucket_hbm, tabt_hbm, out_hbm, bidx_v, tabt_v, row_v):
    c = lax.axis_index("c")
    s = lax.axis_index("s")
    pltpu.sync_copy(bucket_hbm.at[0], bidx_v)
    pltpu.sync_copy(tabt_hbm, tabt_v)
    h_base = jnp.full((_SC_LANES,), s * NUM_BUCKETS, jnp.int32)

    def chunk(t, carry):
        idx = bidx_v[pl.ds((c * _CHUNKS + t) * _SC_LANES, _SC_LANES)]
        vals = plsc.load_gather(tabt_v, [h_base + idx])
        row_v[pl.ds(t * _SC_LANES, _SC_LANES)] = vals
        return carry

    lax.fori_loop(0, _CHUNKS, chunk, 0)
    pltpu.sync_copy(row_v, out_hbm.at[pl.ds(s * TLANE + c * _HALF, _HALF)])


def _expand_body(vpad_ref, out_ref, b_ref, t_ref, sem):
    ib = pl.program_id(0)

    @pl.when(ib == 0)
    def _init():
        for u in range(BI):
            b_ref[:, u, :] = pltpu.roll(vpad_ref[:, :], u, axis=1)

    for m in range(16):
        @pl.when(ib == m)
        def _build_band(m=m):
            t_ref[:, BI * m:BI * (m + 1), :] = pltpu.roll(
                b_ref[:, :, :], BI * m, axis=2
            )

    def _copy(step):
        s0 = (S_I - 1) - step * BI
        b0 = lax.rem(s0, 128)
        q0 = lax.div(s0, 128)
        t0 = lax.div(127 - b0, 8) * 8
        a0 = (q0 + 1) * 128
        return pltpu.make_async_copy(
            t_ref.at[:, pl.ds(t0, BI), pl.ds(a0, S_J)],
            out_ref.at[:, pl.ds(step * BI, BI), :],
            sem,
        )

    _copy(ib).start()

    @pl.when(ib >= NBUF - 1)
    def _drain_one():
        _copy(ib - (NBUF - 1)).wait()

    @pl.when(ib == pl.num_programs(0) - 1)
    def _drain_rest():
        for lag in range(NBUF - 2, -1, -1):
            _copy(ib - lag).wait()


def kernel(i, j, relative_attention_bias):
    delta = (jnp.asarray(j, jnp.int32) - S_J) - (jnp.asarray(i, jnp.int32) - S_I)
    delta = delta.reshape((1,))
    tab_t = relative_attention_bias.T

    bucket = pl.pallas_call(
        _bucketize_body,
        in_specs=[pl.BlockSpec(memory_space=pltpu.SMEM)],
        out_specs=pl.BlockSpec((8, TLANE), lambda: (0, 0)),
        out_shape=jax.ShapeDtypeStruct((8, TLANE), jnp.int32),
    )(delta)

    sc_gather = pl.kernel(
        _sc_gather_body,
        out_type=jax.ShapeDtypeStruct((HEADS * TLANE,), jnp.float32),
        mesh=plsc.VectorSubcoreMesh(core_axis_name="c", subcore_axis_name="s"),
        compiler_params=pltpu.CompilerParams(needs_layout_passes=False),
        scratch_types=[
            pltpu.VMEM((TLANE,), jnp.int32),
            pltpu.VMEM((HEADS * NUM_BUCKETS,), jnp.float32),
            pltpu.VMEM((_HALF,), jnp.float32),
        ],
    )
    vpad = sc_gather(bucket, tab_t.reshape(-1)).reshape(HEADS, TLANE)

    return pl.pallas_call(
        _expand_body,
        grid=(S_I // BI,),
        in_specs=[pl.BlockSpec((HEADS, TLANE), lambda ib: (0, 0))],
        out_specs=pl.BlockSpec(memory_space=pltpu.HBM),
        out_shape=jax.ShapeDtypeStruct((HEADS, S_I, S_J), jnp.float32),
        scratch_shapes=[
            pltpu.VMEM((HEADS, BI, TLANE), jnp.float32),
            pltpu.VMEM((HEADS, 128, TLANE), jnp.float32),
            pltpu.SemaphoreType.DMA,
        ],
    )(vpad)


# speedup vs baseline: 1.0145x; 1.0145x over previous
import jax
import jax.numpy as jnp
from jax import lax
from jax.experimental import pallas as pl
from jax.experimental.pallas import tpu as pltpu
from jax.experimental.pallas import tpu_sc as plsc

NUM_BUCKETS = 32
MAX_DISTANCE = 128
HEADS = 16
S_I = 2048
S_J = 2048

BI = 8
TLANE = 4224
NBUF = 4
_LOG_DENOM = 2.0794415416798357

_SC_LANES = 16
_SC_CORES = 2
_HALF = TLANE // _SC_CORES
_CHUNKS = _HALF // _SC_LANES


def _bucketize_body(delta_ref, out_ref):
    k = lax.broadcasted_iota(jnp.int32, (1, TLANE), 1)
    rel = k - S_I + delta_ref[0]
    n = jnp.maximum(-rel, 0)
    max_exact = NUM_BUCKETS // 2
    is_small = n < max_exact
    safe_n = jnp.maximum(n, 1)
    val_if_large = max_exact + (
        jnp.log(safe_n.astype(jnp.float32) / max_exact)
        / _LOG_DENOM
        * (NUM_BUCKETS - max_exact)
    ).astype(jnp.int32)
    val_if_large = jnp.minimum(val_if_large, NUM_BUCKETS - 1)
    bucket = jnp.where(is_small, n, val_if_large)
    out_ref[:, :] = jnp.broadcast_to(bucket, (8, TLANE))


def _sc_gather_body(bucket_hbm, tabt_hbm, out_hbm, bidx_v, tabt_v, row_v):
    c = lax.axis_index("c")
    s = lax.axis_index("s")
    pltpu.sync_copy(bucket_hbm.at[0], bidx_v)
    pltpu.sync_copy(tabt_hbm, tabt_v)
    h_base = jnp.full((_SC_LANES,), s * NUM_BUCKETS, jnp.int32)

    def chunk(t, carry):
        idx = bidx_v[pl.ds((c * _CHUNKS + t) * _SC_LANES, _SC_LANES)]
        vals = plsc.load_gather(tabt_v, [h_base + idx])
        row_v[pl.ds(t * _SC_LANES, _SC_LANES)] = vals
        return carry

    lax.fori_loop(0, _CHUNKS, chunk, 0)
    pltpu.sync_copy(row_v, out_hbm.at[pl.ds(s * TLANE + c * _HALF, _HALF)])


def _expand_body(vpad_ref, out_ref, b_ref, t_ref, sem):
    ib = pl.program_id(0)

    @pl.when(ib == 0)
    def _init():
        for u in range(BI):
            b_ref[:, u, :] = pltpu.roll(vpad_ref[:, :], u, axis=1)

    for m in range(16):
        @pl.when(ib == m)
        def _build_band(m=m):
            t_ref[:, BI * m:BI * (m + 1), :] = pltpu.roll(
                b_ref[:, :, :], BI * m, axis=2
            )

    def _copy(step):
        s0 = (S_I - 1) - step * BI
        b0 = lax.rem(s0, 128)
        q0 = lax.div(s0, 128)
        t0 = lax.div(127 - b0, 8) * 8
        a0 = (q0 + 1) * 128
        return pltpu.make_async_copy(
            t_ref.at[:, pl.ds(t0, BI), pl.ds(a0, S_J)],
            out_ref.at[:, pl.ds(step * BI, BI), :],
            sem,
        )

    _copy(ib).start()

    @pl.when(ib >= NBUF - 1)
    def _drain_one():
        _copy(ib - (NBUF - 1)).wait()

    @pl.when(ib == pl.num_programs(0) - 1)
    def _drain_rest():
        for lag in range(NBUF - 2, -1, -1):
            _copy(ib - lag).wait()


def kernel(i, j, relative_attention_bias):
    delta = (jnp.asarray(j, jnp.int32) - S_J) - (jnp.asarray(i, jnp.int32) - S_I)
    delta = delta.reshape((1,))
    tab_t = relative_attention_bias.T

    bucket = pl.pallas_call(
        _bucketize_body,
        in_specs=[pl.BlockSpec(memory_space=pltpu.SMEM)],
        out_specs=pl.BlockSpec((8, TLANE), lambda: (0, 0)),
        out_shape=jax.ShapeDtypeStruct((8, TLANE), jnp.int32),
    )(delta)

    sc_gather = pl.kernel(
        _sc_gather_body,
        out_type=jax.ShapeDtypeStruct((HEADS * TLANE,), jnp.float32),
        mesh=plsc.VectorSubcoreMesh(core_axis_name="c", subcore_axis_name="s"),
        compiler_params=pltpu.CompilerParams(needs_layout_passes=False),
        scratch_types=[
            pltpu.VMEM((TLANE,), jnp.int32),
            pltpu.VMEM((HEADS * NUM_BUCKETS,), jnp.float32),
            pltpu.VMEM((_HALF,), jnp.float32),
        ],
    )
    vpad = sc_gather(bucket, tab_t.reshape(-1)).reshape(HEADS, TLANE)

    return pl.pallas_call(
        _expand_body,
        grid=(S_I // BI,),
        in_specs=[pl.BlockSpec((HEADS, TLANE), lambda ib: (0, 0))],
        out_specs=pl.BlockSpec(memory_space=pltpu.HBM),
        out_shape=jax.ShapeDtypeStruct((HEADS, S_I, S_J), jnp.float32),
        scratch_shapes=[
            pltpu.VMEM((HEADS, BI, TLANE), jnp.float32),
            pltpu.VMEM((HEADS, 128, TLANE), jnp.float32),
            pltpu.SemaphoreType.DMA,
        ],
    )(vpad)
